# baseline (speedup 1.0000x reference)
import functools

import jax
import jax.numpy as jnp
from jax import lax
from jax.experimental import pallas as pl
from jax.experimental.pallas import tpu as pltpu
from jax.experimental.pallas import tpu_sc as plsc

_NUM_CLASSES = 10
_B = 16384
_NC, _NS, _L = 1, 16, 16
_NW = _NC * _NS
_BPW = _B // _NW
_CHUNKS = _BPW // _L


@functools.partial(
    pl.kernel,
    out_type=jax.ShapeDtypeStruct((_NUM_CLASSES, _B), jnp.float32),
    mesh=plsc.VectorSubcoreMesh(
        core_axis_name="c", subcore_axis_name="s", num_cores=_NC
    ),
    scratch_types=[
        pltpu.VMEM((_BPW,), jnp.int32),
        pltpu.VMEM((_NUM_CLASSES, _BPW), jnp.float32),
    ],
    compiler_params=pltpu.CompilerParams(
        needs_layout_passes=False, use_tc_tiling_on_sc=True
    ),
)
def _onehot_t_sc(labels_hbm, out_hbm, lab_v, out_v):
    wid = lax.axis_index("s") * _NC + lax.axis_index("c")
    base = wid * _BPW
    pltpu.sync_copy(labels_hbm.at[pl.ds(base, _BPW)], lab_v)

    ones = jnp.ones((_L,), jnp.float32)
    zeros = jnp.zeros((_L,), jnp.float32)

    @plsc.parallel_loop(0, _CHUNKS, unroll=8)
    def _chunk(i):
        lab = lab_v[pl.ds(i * _L, _L)]
        for j in range(_NUM_CLASSES):
            out_v[j, pl.ds(i * _L, _L)] = jnp.where(lab == j, ones, zeros)

    pltpu.sync_copy(out_v, out_hbm.at[:, pl.ds(base, _BPW)])


def kernel(images, labels):
    del images
    return _onehot_t_sc(labels).T


# speedup vs baseline: 1.0227x; 1.0227x over previous
import functools

import jax
import jax.numpy as jnp
from jax import lax
from jax.experimental import pallas as pl
from jax.experimental.pallas import tpu as pltpu
from jax.experimental.pallas import tpu_sc as plsc

_NUM_CLASSES = 10
_B = 16384
_NC, _NS, _L = 2, 16, 16
_NW = _NC * _NS
_BPW = _B // _NW
_CHUNKS = _BPW // _L


@functools.partial(
    pl.kernel,
    out_type=jax.ShapeDtypeStruct((_NUM_CLASSES, _B), jnp.float32),
    mesh=plsc.VectorSubcoreMesh(
        core_axis_name="c", subcore_axis_name="s", num_cores=_NC
    ),
    scratch_types=[
        pltpu.VMEM((_BPW,), jnp.int32),
        pltpu.VMEM((_NUM_CLASSES, _BPW), jnp.float32),
    ],
    compiler_params=pltpu.CompilerParams(
        needs_layout_passes=False, use_tc_tiling_on_sc=True
    ),
)
def _onehot_t_sc(labels_hbm, out_hbm, lab_v, out_v):
    wid = lax.axis_index("s") * _NC + lax.axis_index("c")
    base = wid * _BPW
    pltpu.sync_copy(labels_hbm.at[pl.ds(base, _BPW)], lab_v)

    ones = jnp.ones((_L,), jnp.float32)
    zeros = jnp.zeros((_L,), jnp.float32)

    @plsc.parallel_loop(0, _CHUNKS, unroll=4)
    def _chunk(i):
        lab = lab_v[pl.ds(i * _L, _L)]
        for j in range(_NUM_CLASSES):
            out_v[j, pl.ds(i * _L, _L)] = jnp.where(lab == j, ones, zeros)

    pltpu.sync_copy(out_v, out_hbm.at[:, pl.ds(base, _BPW)])


def kernel(images, labels):
    del images
    return _onehot_t_sc(labels).T


# speedup vs baseline: 1.0705x; 1.0468x over previous
import functools

import jax
import jax.numpy as jnp
from jax import lax
from jax.experimental import pallas as pl
from jax.experimental.pallas import tpu as pltpu
from jax.experimental.pallas import tpu_sc as plsc

_NUM_CLASSES = 10
_B = 16384
_NC, _NS, _L = 1, 16, 16
_NW = _NC * _NS
_BPW = _B // _NW
_CHUNKS = _BPW // _L


@functools.partial(
    pl.kernel,
    out_type=jax.ShapeDtypeStruct((_NUM_CLASSES, _B), jnp.float32),
    mesh=plsc.VectorSubcoreMesh(
        core_axis_name="c", subcore_axis_name="s", num_cores=_NC
    ),
    scratch_types=[
        pltpu.VMEM((_BPW,), jnp.int32),
        pltpu.VMEM((_NUM_CLASSES, _BPW), jnp.float32),
        pltpu.SemaphoreType.DMA,
    ],
    compiler_params=pltpu.CompilerParams(
        needs_layout_passes=False, use_tc_tiling_on_sc=True
    ),
)
def _onehot_t_sc(labels_hbm, out_hbm, lab_v, out_v, sem):
    wid = lax.axis_index("s") * _NC + lax.axis_index("c")
    base = wid * _BPW
    half = _BPW // 2
    pltpu.sync_copy(labels_hbm.at[pl.ds(base, _BPW)], lab_v)

    ones = jnp.ones((_L,), jnp.float32)
    zeros = jnp.zeros((_L,), jnp.float32)

    @plsc.parallel_loop(0, _CHUNKS // 2, unroll=4)
    def _chunk_lo(i):
        lab = lab_v[pl.ds(i * _L, _L)]
        for j in range(_NUM_CLASSES):
            out_v[j, pl.ds(i * _L, _L)] = jnp.where(lab == j, ones, zeros)

    cp_lo = pltpu.make_async_copy(
        out_v.at[:, pl.ds(0, half)], out_hbm.at[:, pl.ds(base, half)], sem
    )
    cp_lo.start()

    @plsc.parallel_loop(_CHUNKS // 2, _CHUNKS, unroll=4)
    def _chunk_hi(i):
        lab = lab_v[pl.ds(i * _L, _L)]
        for j in range(_NUM_CLASSES):
            out_v[j, pl.ds(i * _L, _L)] = jnp.where(lab == j, ones, zeros)

    cp_hi = pltpu.make_async_copy(
        out_v.at[:, pl.ds(half, half)], out_hbm.at[:, pl.ds(base + half, half)], sem
    )
    cp_hi.start()
    cp_lo.wait()
    cp_hi.wait()


def kernel(images, labels):
    del images
    return _onehot_t_sc(labels).T


# speedup vs baseline: 1.0724x; 1.0018x over previous
import functools

import jax
import jax.numpy as jnp
from jax import lax
from jax.experimental import pallas as pl
from jax.experimental.pallas import tpu as pltpu
from jax.experimental.pallas import tpu_sc as plsc

_NUM_CLASSES = 10
_B = 16384
_NC, _NS, _L = 1, 16, 16
_NW = _NC * _NS
_BPW = _B // _NW
_CHUNKS = _BPW // _L


@functools.partial(
    pl.kernel,
    out_type=jax.ShapeDtypeStruct((_NUM_CLASSES, _B), jnp.float32),
    mesh=plsc.VectorSubcoreMesh(
        core_axis_name="c", subcore_axis_name="s", num_cores=_NC
    ),
    scratch_types=[
        pltpu.VMEM((_BPW,), jnp.int32),
        pltpu.VMEM((_NUM_CLASSES, _BPW), jnp.float32),
        pltpu.SemaphoreType.DMA,
    ],
    compiler_params=pltpu.CompilerParams(
        needs_layout_passes=False, use_tc_tiling_on_sc=True
    ),
)
def _onehot_t_sc(labels_hbm, out_hbm, lab_v, out_v, sem):
    wid = lax.axis_index("s") * _NC + lax.axis_index("c")
    base = wid * _BPW
    half = _BPW // 2
    pltpu.sync_copy(labels_hbm.at[pl.ds(base, _BPW)], lab_v)

    ones = jnp.ones((_L,), jnp.float32)
    zeros = jnp.zeros((_L,), jnp.float32)

    @plsc.parallel_loop(0, _CHUNKS // 2, unroll=2)
    def _chunk_lo(i):
        lab = lab_v[pl.ds(i * _L, _L)]
        for j in range(_NUM_CLASSES):
            out_v[j, pl.ds(i * _L, _L)] = jnp.where(lab == j, ones, zeros)

    cp_lo = pltpu.make_async_copy(
        out_v.at[:, pl.ds(0, half)], out_hbm.at[:, pl.ds(base, half)], sem
    )
    cp_lo.start()

    @plsc.parallel_loop(_CHUNKS // 2, _CHUNKS, unroll=2)
    def _chunk_hi(i):
        lab = lab_v[pl.ds(i * _L, _L)]
        for j in range(_NUM_CLASSES):
            out_v[j, pl.ds(i * _L, _L)] = jnp.where(lab == j, ones, zeros)

    cp_hi = pltpu.make_async_copy(
        out_v.at[:, pl.ds(half, half)], out_hbm.at[:, pl.ds(base + half, half)], sem
    )
    cp_hi.start()
    cp_lo.wait()
    cp_hi.wait()


def kernel(images, labels):
    del images
    return _onehot_t_sc(labels).T
